# baseline (speedup 1.0000x reference)
import jax
import jax.numpy as jnp
from jax import lax
from jax.experimental import pallas as pl
from jax.experimental.pallas import tpu as pltpu
from jax.experimental.pallas import tpu_sc as plsc

N_NODES = 100000
NODE_DIM = 128
OUT_DIM = 256
NUM_SEG = 256
NCH = NODE_DIM // 16
ACC_W = NODE_DIM + 16

NW = 32
ROWS_A = 3136
ROWS_B = 3120
NW_A = 10
assert NW_A * ROWS_A + (NW - NW_A) * ROWS_B == N_NODES
CHUNK = 256
NFULL = ROWS_B // CHUNK
GRP_PER_CHUNK = CHUNK // 16
TAIL_A = ROWS_A - NFULL * CHUNK
TAIL_B = ROWS_B - NFULL * CHUNK


def _sc_body(x_hbm, batch_hbm, part_hbm, bidx_v, rowbuf, acc_v, sem0, sem1):
    c = lax.axis_index("c")
    s = lax.axis_index("s")
    wid = s * 2 + c
    is_a = wid < NW_A
    base = jnp.where(is_a, wid * ROWS_A,
                     NW_A * ROWS_A + (wid - NW_A) * ROWS_B)

    pltpu.sync_copy(batch_hbm.at[pl.ds(base, ROWS_B)],
                    bidx_v.at[pl.ds(0, ROWS_B)])

    @pl.when(is_a)
    def _():
        pltpu.sync_copy(batch_hbm.at[pl.ds(base + ROWS_B, ROWS_A - ROWS_B)],
                        bidx_v.at[pl.ds(ROWS_B, ROWS_A - ROWS_B)])

    zeros16 = jnp.zeros((16,), jnp.float32)
    ones16 = jnp.ones((16,), jnp.float32)

    def _zero_row(r, carry):
        for k in range(ACC_W // 16):
            acc_v[pl.ds(r * ACC_W + k * 16, 16)] = zeros16
        return carry

    lax.fori_loop(0, NUM_SEG, _zero_row, 0)

    def _flush(cur, cnt, regs):
        for k in range(NCH):
            plsc.addupdate(acc_v.at[pl.ds(cur * ACC_W + k * 16, 16)], regs[k])
        plsc.addupdate(acc_v.at[pl.ds(cur * ACC_W + NODE_DIM, 16)],
                       jnp.full((16,), 1.0, jnp.float32) * cnt)

    def _group(buf, seg_off, row_off, carry):
        cur, cnt, *regs = carry
        segs = bidx_v[pl.ds(seg_off, 16)]
        last = segs[15]
        fast = last == cur

        tsums = []
        for k in range(NCH):
            t = rowbuf[buf, row_off, pl.ds(k * 16, 16)]
            for j in range(1, 16):
                t = t + rowbuf[buf, row_off + j, pl.ds(k * 16, 16)]
            tsums.append(t)

        @pl.when(jnp.logical_not(fast))
        def _():
            _flush(cur, cnt, regs)
            for j in range(16):
                off = segs[j] * ACC_W
                for k in range(NCH):
                    plsc.addupdate(acc_v.at[pl.ds(off + k * 16, 16)],
                                   rowbuf[buf, row_off + j, pl.ds(k * 16, 16)])
                plsc.addupdate(acc_v.at[pl.ds(off + NODE_DIM, 16)], ones16)

        new_regs = [jnp.where(fast, regs[k] + tsums[k], zeros16)
                    for k in range(NCH)]
        new_cnt = jnp.where(fast, cnt + 16.0, 0.0)
        return (last, new_cnt, *new_regs)

    def _x_copy(ch, buf, sem):
        return pltpu.make_async_copy(
            x_hbm.at[pl.ds(base + ch * CHUNK, CHUNK)], rowbuf.at[buf], sem)

    def _process_chunk(ch, buf, carry):
        def _grp(g, carry):
            carry = _group(buf, ch * CHUNK + g * 32, g * 32, carry)
            return _group(buf, ch * CHUNK + g * 32 + 16, g * 32 + 16, carry)
        return lax.fori_loop(0, GRP_PER_CHUNK // 2, _grp, carry)

    _x_copy(0, 0, sem0).start()
    _x_copy(1, 1, sem1).start()

    cur0 = bidx_v[pl.ds(0, 16)][0]
    carry = (cur0, 0.0, *(zeros16 for _ in range(NCH)))

    def _pair(i, carry):
        ch0 = i * 2
        _x_copy(ch0, 0, sem0).wait()
        carry = _process_chunk(ch0, 0, carry)

        @pl.when(ch0 + 2 < NFULL)
        def _():
            _x_copy(ch0 + 2, 0, sem0).start()

        ch1 = ch0 + 1
        _x_copy(ch1, 1, sem1).wait()
        carry = _process_chunk(ch1, 1, carry)

        @pl.when(ch1 + 2 < NFULL)
        def _():
            _x_copy(ch1 + 2, 1, sem1).start()

        return carry

    carry = lax.fori_loop(0, NFULL // 2, _pair, carry)

    tail_off = NFULL * CHUNK

    @pl.when(is_a)
    def _():
        pltpu.sync_copy(x_hbm.at[pl.ds(base + tail_off, TAIL_A)],
                        rowbuf.at[0, pl.ds(0, TAIL_A)])

    @pl.when(jnp.logical_not(is_a))
    def _():
        pltpu.sync_copy(x_hbm.at[pl.ds(base + tail_off, TAIL_B)],
                        rowbuf.at[0, pl.ds(0, TAIL_B)])

    n_tail_grps = jnp.where(is_a, TAIL_A // 16, TAIL_B // 16)

    def _tail_grp(g, carry):
        return _group(0, tail_off + g * 16, g * 16, carry)

    carry = lax.fori_loop(0, n_tail_grps, _tail_grp, carry)

    cur, cnt, *regs = carry
    _flush(cur, cnt, regs)

    pltpu.sync_copy(acc_v, part_hbm.at[wid])


@jax.jit
def _sc_segment_sums(x, batch_i32):
    mesh = plsc.VectorSubcoreMesh(core_axis_name="c", subcore_axis_name="s")
    return pl.kernel(
        _sc_body,
        out_type=jax.ShapeDtypeStruct((NW, NUM_SEG * ACC_W), jnp.float32),
        mesh=mesh,
        scratch_types=[
            pltpu.VMEM((ROWS_A,), jnp.int32),
            pltpu.VMEM((2, CHUNK, NODE_DIM), jnp.float32),
            pltpu.VMEM((NUM_SEG * ACC_W,), jnp.float32),
            pltpu.SemaphoreType.DMA,
            pltpu.SemaphoreType.DMA,
        ],
    )(x, batch_i32)


def _tc_reduce_body(part_ref, o_ref):
    o_ref[...] = jnp.sum(part_ref[...], axis=0)


def _tc_proj_body(acc_ref, w_ref, b_ref, o_ref):
    acc = acc_ref[...]
    sums = acc[:, :NODE_DIM]
    counts = jnp.maximum(acc[:, NODE_DIM], 1.0)
    pooled = sums / counts[:, None]
    o_ref[...] = (
        jnp.dot(pooled, w_ref[...], preferred_element_type=jnp.float32)
        + b_ref[...]
    )


@jax.jit
def _tc_project(part, W, b2d):
    flat = pl.pallas_call(
        _tc_reduce_body,
        out_shape=jax.ShapeDtypeStruct((NUM_SEG * ACC_W,), jnp.float32),
    )(part)
    acc = flat.reshape(NUM_SEG, ACC_W)
    return pl.pallas_call(
        _tc_proj_body,
        out_shape=jax.ShapeDtypeStruct((NUM_SEG, OUT_DIM), jnp.float32),
    )(acc, W, b2d)


def kernel(x, batch, W, b):
    part = _sc_segment_sums(x, batch.astype(jnp.int32))
    return _tc_project(part, W, b.reshape(1, OUT_DIM))


# speedup vs baseline: 1.3809x; 1.3809x over previous
import jax
import jax.numpy as jnp
from jax import lax
from jax.experimental import pallas as pl
from jax.experimental.pallas import tpu as pltpu
from jax.experimental.pallas import tpu_sc as plsc

N_NODES = 100000
NODE_DIM = 128
OUT_DIM = 256
NUM_SEG = 256
NCH = NODE_DIM // 16
ACC_W = NODE_DIM + 16

NW = 32
ROWS_A = 3136
ROWS_B = 3120
NW_A = 10
assert NW_A * ROWS_A + (NW - NW_A) * ROWS_B == N_NODES
CHUNK = 256
NFULL = ROWS_B // CHUNK
GRP_PER_CHUNK = CHUNK // 16
TAIL_A = ROWS_A - NFULL * CHUNK
TAIL_B = ROWS_B - NFULL * CHUNK


def _sc_body(x_hbm, batch_hbm, part_hbm, bidx_v, rowbuf, acc_v, sem0, sem1):
    c = lax.axis_index("c")
    s = lax.axis_index("s")
    wid = s * 2 + c
    is_a = wid < NW_A
    base = jnp.where(is_a, wid * ROWS_A,
                     NW_A * ROWS_A + (wid - NW_A) * ROWS_B)

    pltpu.sync_copy(batch_hbm.at[pl.ds(base, ROWS_B)],
                    bidx_v.at[pl.ds(0, ROWS_B)])

    @pl.when(is_a)
    def _():
        pltpu.sync_copy(batch_hbm.at[pl.ds(base + ROWS_B, ROWS_A - ROWS_B)],
                        bidx_v.at[pl.ds(ROWS_B, ROWS_A - ROWS_B)])

    zeros16 = jnp.zeros((16,), jnp.float32)
    ones16 = jnp.ones((16,), jnp.float32)

    def _zero_row(r, carry):
        for k in range(ACC_W // 16):
            acc_v[pl.ds(r * ACC_W + k * 16, 16)] = zeros16
        return carry

    lax.fori_loop(0, NUM_SEG, _zero_row, 0)

    def _flush(cur, cnt, regs):
        for k in range(NCH):
            plsc.addupdate(acc_v.at[pl.ds(cur * ACC_W + k * 16, 16)], regs[k])
        plsc.addupdate(acc_v.at[pl.ds(cur * ACC_W + NODE_DIM, 16)],
                       jnp.full((16,), 1.0, jnp.float32) * cnt)

    def _group(buf, seg_off, row_off, carry):
        cur, cnt, *regs = carry
        segs = bidx_v[pl.ds(seg_off, 16)]
        last = segs[15]
        fast = last == cur

        tsums = []
        for k in range(NCH):
            t = rowbuf[buf, row_off, pl.ds(k * 16, 16)]
            for j in range(1, 16):
                t = t + rowbuf[buf, row_off + j, pl.ds(k * 16, 16)]
            tsums.append(t)

        @pl.when(jnp.logical_not(fast))
        def _():
            _flush(cur, cnt, regs)
            for j in range(16):
                off = segs[j] * ACC_W
                for k in range(NCH):
                    plsc.addupdate(acc_v.at[pl.ds(off + k * 16, 16)],
                                   rowbuf[buf, row_off + j, pl.ds(k * 16, 16)])
                plsc.addupdate(acc_v.at[pl.ds(off + NODE_DIM, 16)], ones16)

        new_regs = [jnp.where(fast, regs[k] + tsums[k], zeros16)
                    for k in range(NCH)]
        new_cnt = jnp.where(fast, cnt + 16.0, 0.0)
        return (last, new_cnt, *new_regs)

    def _x_copy(ch, buf, sem):
        return pltpu.make_async_copy(
            x_hbm.at[pl.ds(base + ch * CHUNK, CHUNK)], rowbuf.at[buf], sem)

    def _process_chunk(ch, buf, carry):
        def _grp(g, carry):
            return _group(buf, ch * CHUNK + g * 16, g * 16, carry)
        return lax.fori_loop(0, GRP_PER_CHUNK, _grp, carry)

    _x_copy(0, 0, sem0).start()
    _x_copy(1, 1, sem1).start()

    cur0 = bidx_v[pl.ds(0, 16)][0]
    carry = (cur0, 0.0, *(zeros16 for _ in range(NCH)))

    def _pair(i, carry):
        ch0 = i * 2
        _x_copy(ch0, 0, sem0).wait()
        carry = _process_chunk(ch0, 0, carry)

        @pl.when(ch0 + 2 < NFULL)
        def _():
            _x_copy(ch0 + 2, 0, sem0).start()

        ch1 = ch0 + 1
        _x_copy(ch1, 1, sem1).wait()
        carry = _process_chunk(ch1, 1, carry)

        @pl.when(ch1 + 2 < NFULL)
        def _():
            _x_copy(ch1 + 2, 1, sem1).start()

        return carry

    carry = lax.fori_loop(0, NFULL // 2, _pair, carry)

    tail_off = NFULL * CHUNK

    @pl.when(is_a)
    def _():
        pltpu.sync_copy(x_hbm.at[pl.ds(base + tail_off, TAIL_A)],
                        rowbuf.at[0, pl.ds(0, TAIL_A)])

    @pl.when(jnp.logical_not(is_a))
    def _():
        pltpu.sync_copy(x_hbm.at[pl.ds(base + tail_off, TAIL_B)],
                        rowbuf.at[0, pl.ds(0, TAIL_B)])

    n_tail_grps = jnp.where(is_a, TAIL_A // 16, TAIL_B // 16)

    def _tail_grp(g, carry):
        return _group(0, tail_off + g * 16, g * 16, carry)

    carry = lax.fori_loop(0, n_tail_grps, _tail_grp, carry)

    cur, cnt, *regs = carry
    _flush(cur, cnt, regs)

    pltpu.sync_copy(acc_v, part_hbm.at[wid])


@jax.jit
def _sc_segment_sums(x, batch_i32):
    mesh = plsc.VectorSubcoreMesh(core_axis_name="c", subcore_axis_name="s")
    return pl.kernel(
        _sc_body,
        out_type=jax.ShapeDtypeStruct((NW, NUM_SEG * ACC_W), jnp.float32),
        mesh=mesh,
        scratch_types=[
            pltpu.VMEM((ROWS_A,), jnp.int32),
            pltpu.VMEM((2, CHUNK, NODE_DIM), jnp.float32),
            pltpu.VMEM((NUM_SEG * ACC_W,), jnp.float32),
            pltpu.SemaphoreType.DMA,
            pltpu.SemaphoreType.DMA,
        ],
    )(x, batch_i32)


def _tc_reduce_body(part_ref, o_ref):
    o_ref[...] = jnp.sum(part_ref[...], axis=0)


def _tc_proj_body(acc_ref, w_ref, b_ref, o_ref):
    acc = acc_ref[...]
    sums = acc[:, :NODE_DIM]
    counts = jnp.maximum(acc[:, NODE_DIM], 1.0)
    pooled = sums / counts[:, None]
    o_ref[...] = (
        jnp.dot(pooled, w_ref[...], preferred_element_type=jnp.float32)
        + b_ref[...]
    )


@jax.jit
def _tc_project(part, W, b2d):
    flat = pl.pallas_call(
        _tc_reduce_body,
        out_shape=jax.ShapeDtypeStruct((NUM_SEG * ACC_W,), jnp.float32),
    )(part)
    acc = flat.reshape(NUM_SEG, ACC_W)
    return pl.pallas_call(
        _tc_proj_body,
        out_shape=jax.ShapeDtypeStruct((NUM_SEG, OUT_DIM), jnp.float32),
    )(acc, W, b2d)


def kernel(x, batch, W, b):
    part = _sc_segment_sums(x, batch.astype(jnp.int32))
    return _tc_project(part, W, b.reshape(1, OUT_DIM))
